# Initial kernel scaffold; baseline (speedup 1.0000x reference)
import jax
import jax.numpy as jnp
from jax.experimental import pallas as pl


def kernel(x, edge_index, batch, W1, b1, W2, b2, M1, mb1, M2, mb2):
    raise NotImplementedError("write your pallas kernel here")


# speedup vs baseline: 5.2665x; 5.2665x over previous
import functools

import jax
import jax.numpy as jnp
from jax import lax
from jax.experimental import pallas as pl
from jax.experimental.pallas import tpu as pltpu
from jax.experimental.pallas import tpu_sc as plsc

N = 10000
D = 128
G = 64
E = 320000

NC = 2
NS = 16
NW = NC * NS

NP = 10240
HALF = NP // 2
ACCR = HALF + 8
K = 128
CHS = 160
EPS = NS * CHS * K
RPS = HALF // NS
NID = 5
IDW = RPS // NID
PADN = N


@functools.lru_cache(maxsize=None)
def _sc_mesh():
    return plsc.VectorSubcoreMesh(
        core_axis_name="c", subcore_axis_name="s",
        num_cores=NC, num_subcores=NS)


def _fill_identity(idz, s, iota16):
    def fill_idz(r, _):
        row0 = s * RPS + r * IDW
        for k16 in range(IDW // 16):
            idz[r, pl.ds(k16 * 16, 16)] = iota16 + (row0 + k16 * 16)
        return 0
    lax.fori_loop(0, NID, fill_idz, 0)


def _remap_dst(dst_v, base):
    def remap(j, _):
        for k8 in range(K // 16):
            d = dst_v[j, pl.ds(k8 * 16, 16)]
            l = d - base
            ok = (l >= 0) & (l < HALF)
            dst_v[j, pl.ds(k8 * 16, 16)] = jnp.where(ok, l, HALF)
        return 0
    lax.fori_loop(0, CHS, remap, 0)


@functools.lru_cache(maxsize=None)
def _get_deg_sc():
    return functools.partial(
        pl.kernel,
        out_type=jax.ShapeDtypeStruct((NC, HALF, 16), jnp.float32),
        mesh=_sc_mesh(),
        scratch_types=[
            pltpu.VMEM((CHS, K), jnp.int32),
            pltpu.VMEM((K, 16), jnp.float32),
            pltpu.VMEM((IDW, 16), jnp.float32),
            pltpu.VMEM((NID, IDW), jnp.int32),
            pltpu.VMEM((IDW, 16), jnp.float32),
            pltpu.VMEM_SHARED((ACCR, 16), jnp.float32),
        ],
    )(_deg_sc_body)


def _deg_sc_body(dst_hbm, out_hbm, dst_v, ones_v, zero_v, idz, bounce, acc_sh):
    c = lax.axis_index("c")
    s = lax.axis_index("s")
    base = c * HALF

    ones16 = jnp.full((16,), 1.0, jnp.float32)
    zeros16 = jnp.zeros((16,), jnp.float32)
    iota16 = lax.iota(jnp.int32, 16)

    def fill_ones(i, _):
        ones_v[i, :] = ones16
        return 0
    lax.fori_loop(0, K, fill_ones, 0)
    for i in range(IDW):
        zero_v[i, :] = zeros16
    _fill_identity(idz, s, iota16)

    def zinit(r, _):
        pltpu.sync_copy(zero_v, acc_sh.at[idz.at[r]])
        return 0
    lax.fori_loop(0, NID, zinit, 0)
    plsc.subcore_barrier()

    pltpu.sync_copy(dst_hbm.at[s], dst_v)
    _remap_dst(dst_v, base)

    def body(j, _):
        pltpu.sync_copy(ones_v, acc_sh.at[dst_v.at[j]], add=True)
        return 0
    lax.fori_loop(0, CHS, body, 0)
    plsc.subcore_barrier()

    def cout(r, _):
        pltpu.sync_copy(acc_sh.at[idz.at[r]], bounce)
        pltpu.sync_copy(bounce,
                        out_hbm.at[c, pl.ds(s * RPS + r * IDW, IDW)])
        return 0
    lax.fori_loop(0, NID, cout, 0)


@functools.lru_cache(maxsize=None)
def _get_scatter_sc():
    return functools.partial(
        pl.kernel,
        out_type=jax.ShapeDtypeStruct((NC, HALF, D), jnp.float32),
        mesh=_sc_mesh(),
        scratch_types=[
            pltpu.VMEM((CHS, K), jnp.int32),
            pltpu.VMEM((CHS, K), jnp.int32),
            pltpu.VMEM((K, D), jnp.float32),
            pltpu.VMEM((IDW, D), jnp.float32),
            pltpu.VMEM((NID, IDW), jnp.int32),
            pltpu.VMEM((IDW, D), jnp.float32),
            pltpu.VMEM_SHARED((ACCR, D), jnp.float32),
            pltpu.SemaphoreType.DMA,
        ],
    )(_scatter_sc_body)


def _scatter_sc_body(y_hbm, src_hbm, dst_hbm, out_hbm,
                     src_v, dst_v, rows_v, zero_v, idz, bounce, acc_sh, sem):
    c = lax.axis_index("c")
    s = lax.axis_index("s")
    base = c * HALF

    zeros16 = jnp.zeros((16,), jnp.float32)
    iota16 = lax.iota(jnp.int32, 16)

    def fill_zero(i, _):
        for k8 in range(D // 16):
            zero_v[i, pl.ds(k8 * 16, 16)] = zeros16
        return 0
    lax.fori_loop(0, IDW, fill_zero, 0)
    _fill_identity(idz, s, iota16)

    def zinit(r, _):
        pltpu.sync_copy(zero_v, acc_sh.at[idz.at[r]])
        return 0
    lax.fori_loop(0, NID, zinit, 0)
    plsc.subcore_barrier()

    pltpu.sync_copy(src_hbm.at[s], src_v)
    pltpu.sync_copy(dst_hbm.at[s], dst_v)
    _remap_dst(dst_v, base)

    def body(j, _):
        pltpu.async_copy(y_hbm.at[src_v.at[j]], rows_v, sem).wait()
        pltpu.sync_copy(rows_v, acc_sh.at[dst_v.at[j]], add=True)
        return 0
    lax.fori_loop(0, CHS, body, 0)
    plsc.subcore_barrier()

    def cout(r, _):
        pltpu.sync_copy(acc_sh.at[idz.at[r]], bounce)
        pltpu.sync_copy(bounce,
                        out_hbm.at[c, pl.ds(s * RPS + r * IDW, IDW)])
        return 0
    lax.fori_loop(0, NID, cout, 0)


BR = 640
BR5 = 320
_NBH = HALF // BR
_NBH5 = HALF // BR5


def _dis_block(deg_ref):
    return lax.rsqrt(deg_ref[0, :, 0] + 1.0)


def _k1_body(deg_ref, x_ref, w_ref, y_ref):
    dis = _dis_block(deg_ref)
    xw = jnp.dot(x_ref[...], w_ref[...], preferred_element_type=jnp.float32)
    y_ref[...] = dis[:, None] * xw


def _k1(deg2, x_pad, W1):
    return pl.pallas_call(
        _k1_body,
        grid=(NP // BR,),
        in_specs=[
            pl.BlockSpec((1, BR, 16), lambda i: (i // _NBH, i % _NBH, 0)),
            pl.BlockSpec((BR, D), lambda i: (i, 0)),
            pl.BlockSpec((D, D), lambda i: (0, 0)),
        ],
        out_specs=pl.BlockSpec((BR, D), lambda i: (i, 0)),
        out_shape=jax.ShapeDtypeStruct((NP, D), jnp.float32),
    )(deg2, x_pad, W1)


def _k3_body(deg_ref, acc_ref, y_ref, b_ref, w_ref, out_ref):
    dis = _dis_block(deg_ref)
    t = acc_ref[0] + y_ref[...]
    h = jnp.maximum(dis[:, None] * t + b_ref[...], 0.0)
    hw = jnp.dot(h, w_ref[...], preferred_element_type=jnp.float32)
    out_ref[...] = dis[:, None] * hw


def _k3(deg2, acc, y1, b1r, W2):
    return pl.pallas_call(
        _k3_body,
        grid=(NP // BR,),
        in_specs=[
            pl.BlockSpec((1, BR, 16), lambda i: (i // _NBH, i % _NBH, 0)),
            pl.BlockSpec((1, BR, D), lambda i: (i // _NBH, i % _NBH, 0)),
            pl.BlockSpec((BR, D), lambda i: (i, 0)),
            pl.BlockSpec((1, D), lambda i: (0, 0)),
            pl.BlockSpec((D, D), lambda i: (0, 0)),
        ],
        out_specs=pl.BlockSpec((BR, D), lambda i: (i, 0)),
        out_shape=jax.ShapeDtypeStruct((NP, D), jnp.float32),
    )(deg2, acc, y1, b1r, W2)


def _k5_body(deg_ref, acc_ref, y_ref, b_ref, batch_ref,
             m1_ref, mb1_ref, m2_ref, mb2_ref, out_ref, pooled):
    i = pl.program_id(0)
    dis = _dis_block(deg_ref)
    t = acc_ref[0] + y_ref[...]
    h2 = jnp.maximum(dis[:, None] * t + b_ref[...], 0.0)

    b = batch_ref[0, 0, :]
    gid = lax.broadcasted_iota(jnp.int32, (BR5, 128), 1)
    oh = (b[:, None] == gid).astype(jnp.float32)
    contrib = lax.dot_general(oh, h2, (((0,), (0,)), ((), ())),
                              preferred_element_type=jnp.float32)

    @pl.when(i == 0)
    def _():
        pooled[...] = contrib

    @pl.when(i > 0)
    def _():
        pooled[...] = pooled[...] + contrib

    @pl.when(i == pl.num_programs(0) - 1)
    def _():
        z = jnp.maximum(
            jnp.dot(pooled[0:G, :], m1_ref[...],
                    preferred_element_type=jnp.float32) + mb1_ref[...], 0.0)
        out_ref[...] = jnp.dot(
            z, m2_ref[...], preferred_element_type=jnp.float32) + mb2_ref[...]


def _k5(deg2, acc, y2, b2r, batch3, M1, mb1r, M2, mb2r):
    return pl.pallas_call(
        _k5_body,
        grid=(NP // BR5,),
        in_specs=[
            pl.BlockSpec((1, BR5, 16), lambda i: (i // _NBH5, i % _NBH5, 0)),
            pl.BlockSpec((1, BR5, D), lambda i: (i // _NBH5, i % _NBH5, 0)),
            pl.BlockSpec((BR5, D), lambda i: (i, 0)),
            pl.BlockSpec((1, D), lambda i: (0, 0)),
            pl.BlockSpec((1, 1, BR5), lambda i: (i, 0, 0)),
            pl.BlockSpec((D, D), lambda i: (0, 0)),
            pl.BlockSpec((1, D), lambda i: (0, 0)),
            pl.BlockSpec((D, D), lambda i: (0, 0)),
            pl.BlockSpec((1, D), lambda i: (0, 0)),
        ],
        out_specs=pl.BlockSpec((G, D), lambda i: (0, 0)),
        out_shape=jax.ShapeDtypeStruct((G, D), jnp.float32),
        scratch_shapes=[pltpu.VMEM((128, D), jnp.float32)],
    )(deg2, acc, y2, b2r, batch3, M1, mb1r, M2, mb2r)


@jax.jit
def kernel(x, edge_index, batch, W1, b1, W2, b2, M1, mb1, M2, mb2):
    x_pad = jnp.zeros((NP, D), jnp.float32).at[:N].set(x)
    pads = jnp.full((EPS - E,), PADN, jnp.int32)
    src3 = jnp.concatenate([edge_index[0], pads]).reshape(NS, CHS, K)
    dst3 = jnp.concatenate([edge_index[1], pads]).reshape(NS, CHS, K)
    batch3 = jnp.concatenate(
        [batch, jnp.full((NP - N,), G, jnp.int32)]).reshape(NP // BR5, 1, BR5)
    b1r = b1.reshape(1, D)
    b2r = b2.reshape(1, D)
    mb1r = mb1.reshape(1, D)
    mb2r = mb2.reshape(1, D)

    deg2 = _get_deg_sc()(dst3)
    y1 = _k1(deg2, x_pad, W1)
    acc1 = _get_scatter_sc()(y1, src3, dst3)
    y2 = _k3(deg2, acc1, y1, b1r, W2)
    acc2 = _get_scatter_sc()(y2, src3, dst3)
    return _k5(deg2, acc2, y2, b2r, batch3, M1, mb1r, M2, mb2r)


# speedup vs baseline: 5.4791x; 1.0404x over previous
import functools

import jax
import jax.numpy as jnp
from jax import lax
from jax.experimental import pallas as pl
from jax.experimental.pallas import tpu as pltpu
from jax.experimental.pallas import tpu_sc as plsc

N = 10000
D = 128
G = 64
E = 320000

NC = 2
NS = 16
NW = NC * NS

NP = 10240
HALF = NP // 2
ACCR = HALF + 8
K = 128
CHS = 160
EPS = NS * CHS * K
RPS = HALF // NS
NID = 5
IDW = RPS // NID
PADN = N


@functools.lru_cache(maxsize=None)
def _sc_mesh():
    return plsc.VectorSubcoreMesh(
        core_axis_name="c", subcore_axis_name="s",
        num_cores=NC, num_subcores=NS)


def _fill_identity(idz, s, iota16):
    def fill_idz(r, _):
        row0 = s * RPS + r * IDW
        for k16 in range(IDW // 16):
            idz[r, pl.ds(k16 * 16, 16)] = iota16 + (row0 + k16 * 16)
        return 0
    lax.fori_loop(0, NID, fill_idz, 0)


def _remap_dst(dst_v, base, nchunks=CHS):
    def remap(j, _):
        for k8 in range(K // 16):
            d = dst_v[j, pl.ds(k8 * 16, 16)]
            l = d - base
            ok = (l >= 0) & (l < HALF)
            dst_v[j, pl.ds(k8 * 16, 16)] = jnp.where(ok, l, HALF)
        return 0
    lax.fori_loop(0, nchunks, remap, 0)


@functools.lru_cache(maxsize=None)
def _get_deg_sc():
    return functools.partial(
        pl.kernel,
        out_type=jax.ShapeDtypeStruct((NC, HALF, 16), jnp.float32),
        mesh=_sc_mesh(),
        scratch_types=[
            pltpu.VMEM((CHS, K), jnp.int32),
            pltpu.VMEM((K, 16), jnp.float32),
            pltpu.VMEM((IDW, 16), jnp.float32),
            pltpu.VMEM((NID, IDW), jnp.int32),
            pltpu.VMEM((IDW, 16), jnp.float32),
            pltpu.VMEM_SHARED((ACCR, 16), jnp.float32),
        ],
    )(_deg_sc_body)


def _deg_sc_body(dst_hbm, out_hbm, dst_v, ones_v, zero_v, idz, bounce, acc_sh):
    c = lax.axis_index("c")
    s = lax.axis_index("s")
    base = c * HALF

    ones16 = jnp.full((16,), 1.0, jnp.float32)
    zeros16 = jnp.zeros((16,), jnp.float32)
    iota16 = lax.iota(jnp.int32, 16)

    def fill_ones(i, _):
        ones_v[i, :] = ones16
        return 0
    lax.fori_loop(0, K, fill_ones, 0)
    for i in range(IDW):
        zero_v[i, :] = zeros16
    _fill_identity(idz, s, iota16)

    def zinit(r, _):
        pltpu.sync_copy(zero_v, acc_sh.at[idz.at[r]])
        return 0
    lax.fori_loop(0, NID, zinit, 0)
    plsc.subcore_barrier()

    pltpu.sync_copy(dst_hbm.at[s], dst_v)
    _remap_dst(dst_v, base)

    def body(j, _):
        pltpu.sync_copy(ones_v, acc_sh.at[dst_v.at[j]], add=True)
        return 0
    lax.fori_loop(0, CHS, body, 0)
    plsc.subcore_barrier()

    def cout(r, _):
        pltpu.sync_copy(acc_sh.at[idz.at[r]], bounce)
        pltpu.sync_copy(bounce,
                        out_hbm.at[c, pl.ds(s * RPS + r * IDW, IDW)])
        return 0
    lax.fori_loop(0, NID, cout, 0)


@functools.lru_cache(maxsize=None)
def _get_scatter_sc():
    return functools.partial(
        pl.kernel,
        out_type=jax.ShapeDtypeStruct((NC, HALF, D), jnp.float32),
        mesh=_sc_mesh(),
        scratch_types=[
            pltpu.VMEM((CHS // 2, K), jnp.int32),
            pltpu.VMEM((CHS // 2, K), jnp.int32),
            pltpu.VMEM((K, D), jnp.float32),
            pltpu.VMEM((K, D), jnp.float32),
            pltpu.VMEM((IDW, D), jnp.float32),
            pltpu.VMEM((NID, IDW), jnp.int32),
            pltpu.VMEM((IDW, D), jnp.float32),
            pltpu.VMEM_SHARED((ACCR, D), jnp.float32),
            pltpu.SemaphoreType.DMA,
            pltpu.SemaphoreType.DMA,
        ],
    )(_scatter_sc_body)


def _scatter_sc_body(y_hbm, src_hbm, dst_hbm, out_hbm,
                     src_v, dst_v, rows_a, rows_b, zero_v, idz, bounce,
                     acc_sh, sga, sgb):
    c = lax.axis_index("c")
    s = lax.axis_index("s")
    base = c * HALF

    zeros16 = jnp.zeros((16,), jnp.float32)
    iota16 = lax.iota(jnp.int32, 16)

    def fill_zero(i, _):
        for k8 in range(D // 16):
            zero_v[i, pl.ds(k8 * 16, 16)] = zeros16
        return 0
    lax.fori_loop(0, IDW, fill_zero, 0)
    _fill_identity(idz, s, iota16)

    def zinit(r, _):
        pltpu.sync_copy(zero_v, acc_sh.at[idz.at[r]])
        return 0
    lax.fori_loop(0, NID, zinit, 0)
    plsc.subcore_barrier()

    CSEG = CHS // 2
    for seg in range(2):
        pltpu.sync_copy(src_hbm.at[s, pl.ds(seg * CSEG, CSEG)], src_v)
        pltpu.sync_copy(dst_hbm.at[s, pl.ds(seg * CSEG, CSEG)], dst_v)
        _remap_dst(dst_v, base, CSEG)

        pltpu.async_copy(y_hbm.at[src_v.at[0]], rows_a, sga)

        def body(i, _):
            j = 2 * i
            jb = j + 1
            pltpu.async_copy(y_hbm.at[src_v.at[jb]], rows_b, sgb)
            pltpu.make_async_copy(y_hbm.at[src_v.at[j]], rows_a, sga).wait()
            pltpu.sync_copy(rows_a, acc_sh.at[dst_v.at[j]], add=True)
            ja = jnp.where(jb + 1 < CSEG, jb + 1, j)
            pltpu.async_copy(y_hbm.at[src_v.at[ja]], rows_a, sga)
            pltpu.make_async_copy(y_hbm.at[src_v.at[jb]], rows_b, sgb).wait()
            pltpu.sync_copy(rows_b, acc_sh.at[dst_v.at[jb]], add=True)
            return 0
        lax.fori_loop(0, CSEG // 2, body, 0)
        pltpu.make_async_copy(y_hbm.at[src_v.at[0]], rows_a, sga).wait()
    plsc.subcore_barrier()

    def cout(r, _):
        pltpu.sync_copy(acc_sh.at[idz.at[r]], bounce)
        pltpu.sync_copy(bounce,
                        out_hbm.at[c, pl.ds(s * RPS + r * IDW, IDW)])
        return 0
    lax.fori_loop(0, NID, cout, 0)


BR = 640
BR5 = 320
_NBH = HALF // BR
_NBH5 = HALF // BR5


def _dis_block(deg_ref):
    return lax.rsqrt(deg_ref[0, :, 0] + 1.0)


def _k1_body(deg_ref, x_ref, w_ref, y_ref):
    dis = _dis_block(deg_ref)
    xw = jnp.dot(x_ref[...], w_ref[...], preferred_element_type=jnp.float32)
    y_ref[...] = dis[:, None] * xw


def _k1(deg2, x_pad, W1):
    return pl.pallas_call(
        _k1_body,
        grid=(NP // BR,),
        in_specs=[
            pl.BlockSpec((1, BR, 16), lambda i: (i // _NBH, i % _NBH, 0)),
            pl.BlockSpec((BR, D), lambda i: (i, 0)),
            pl.BlockSpec((D, D), lambda i: (0, 0)),
        ],
        out_specs=pl.BlockSpec((BR, D), lambda i: (i, 0)),
        out_shape=jax.ShapeDtypeStruct((NP, D), jnp.float32),
    )(deg2, x_pad, W1)


def _k3_body(deg_ref, acc_ref, y_ref, b_ref, w_ref, out_ref):
    dis = _dis_block(deg_ref)
    t = acc_ref[0] + y_ref[...]
    h = jnp.maximum(dis[:, None] * t + b_ref[...], 0.0)
    hw = jnp.dot(h, w_ref[...], preferred_element_type=jnp.float32)
    out_ref[...] = dis[:, None] * hw


def _k3(deg2, acc, y1, b1r, W2):
    return pl.pallas_call(
        _k3_body,
        grid=(NP // BR,),
        in_specs=[
            pl.BlockSpec((1, BR, 16), lambda i: (i // _NBH, i % _NBH, 0)),
            pl.BlockSpec((1, BR, D), lambda i: (i // _NBH, i % _NBH, 0)),
            pl.BlockSpec((BR, D), lambda i: (i, 0)),
            pl.BlockSpec((1, D), lambda i: (0, 0)),
            pl.BlockSpec((D, D), lambda i: (0, 0)),
        ],
        out_specs=pl.BlockSpec((BR, D), lambda i: (i, 0)),
        out_shape=jax.ShapeDtypeStruct((NP, D), jnp.float32),
    )(deg2, acc, y1, b1r, W2)


def _k5_body(deg_ref, acc_ref, y_ref, b_ref, batch_ref,
             m1_ref, mb1_ref, m2_ref, mb2_ref, out_ref, pooled):
    i = pl.program_id(0)
    dis = _dis_block(deg_ref)
    t = acc_ref[0] + y_ref[...]
    h2 = jnp.maximum(dis[:, None] * t + b_ref[...], 0.0)

    b = batch_ref[0, 0, :]
    gid = lax.broadcasted_iota(jnp.int32, (BR5, 128), 1)
    oh = (b[:, None] == gid).astype(jnp.float32)
    contrib = lax.dot_general(oh, h2, (((0,), (0,)), ((), ())),
                              preferred_element_type=jnp.float32)

    @pl.when(i == 0)
    def _():
        pooled[...] = contrib

    @pl.when(i > 0)
    def _():
        pooled[...] = pooled[...] + contrib

    @pl.when(i == pl.num_programs(0) - 1)
    def _():
        z = jnp.maximum(
            jnp.dot(pooled[0:G, :], m1_ref[...],
                    preferred_element_type=jnp.float32) + mb1_ref[...], 0.0)
        out_ref[...] = jnp.dot(
            z, m2_ref[...], preferred_element_type=jnp.float32) + mb2_ref[...]


def _k5(deg2, acc, y2, b2r, batch3, M1, mb1r, M2, mb2r):
    return pl.pallas_call(
        _k5_body,
        grid=(NP // BR5,),
        in_specs=[
            pl.BlockSpec((1, BR5, 16), lambda i: (i // _NBH5, i % _NBH5, 0)),
            pl.BlockSpec((1, BR5, D), lambda i: (i // _NBH5, i % _NBH5, 0)),
            pl.BlockSpec((BR5, D), lambda i: (i, 0)),
            pl.BlockSpec((1, D), lambda i: (0, 0)),
            pl.BlockSpec((1, 1, BR5), lambda i: (i, 0, 0)),
            pl.BlockSpec((D, D), lambda i: (0, 0)),
            pl.BlockSpec((1, D), lambda i: (0, 0)),
            pl.BlockSpec((D, D), lambda i: (0, 0)),
            pl.BlockSpec((1, D), lambda i: (0, 0)),
        ],
        out_specs=pl.BlockSpec((G, D), lambda i: (0, 0)),
        out_shape=jax.ShapeDtypeStruct((G, D), jnp.float32),
        scratch_shapes=[pltpu.VMEM((128, D), jnp.float32)],
    )(deg2, acc, y2, b2r, batch3, M1, mb1r, M2, mb2r)


@jax.jit
def kernel(x, edge_index, batch, W1, b1, W2, b2, M1, mb1, M2, mb2):
    x_pad = jnp.zeros((NP, D), jnp.float32).at[:N].set(x)
    pads = jnp.full((EPS - E,), PADN, jnp.int32)
    src3 = jnp.concatenate([edge_index[0], pads]).reshape(NS, CHS, K)
    dst3 = jnp.concatenate([edge_index[1], pads]).reshape(NS, CHS, K)
    batch3 = jnp.concatenate(
        [batch, jnp.full((NP - N,), G, jnp.int32)]).reshape(NP // BR5, 1, BR5)
    b1r = b1.reshape(1, D)
    b2r = b2.reshape(1, D)
    mb1r = mb1.reshape(1, D)
    mb2r = mb2.reshape(1, D)

    deg2 = _get_deg_sc()(dst3)
    y1 = _k1(deg2, x_pad, W1)
    acc1 = _get_scatter_sc()(y1, src3, dst3)
    y2 = _k3(deg2, acc1, y1, b1r, W2)
    acc2 = _get_scatter_sc()(y2, src3, dst3)
    return _k5(deg2, acc2, y2, b2r, batch3, M1, mb1r, M2, mb2r)
